# Initial kernel scaffold; baseline (speedup 1.0000x reference)
import jax
import jax.numpy as jnp
from jax.experimental import pallas as pl


def kernel(nodes, neigh_idx, features, W1, b1, W2, b2):
    raise NotImplementedError("write your pallas kernel here")


# speedup vs baseline: 1.5616x; 1.5616x over previous
import functools

import jax
import jax.numpy as jnp
from jax import lax
from jax.experimental import pallas as pl
from jax.experimental.pallas import tpu as pltpu
from jax.experimental.pallas import tpu_sc as plsc

B = 16384
D = 128
S = 10
L = 16
NC = 2
NS = 16
NW = NC * NS
BPW = B // NW
G = 8
GI = G * S
NGRP = BPW // G
ACC_E = 64
SELF_CH = 128

_mesh = plsc.VectorSubcoreMesh(core_axis_name="c", subcore_axis_name="s")


@functools.partial(
    pl.kernel,
    out_type=(
        jax.ShapeDtypeStruct((B, D), jnp.float32),
        jax.ShapeDtypeStruct((B, D), jnp.float32),
    ),
    mesh=_mesh,
    scratch_types=[
        pltpu.VMEM((BPW,), jnp.int32),
        pltpu.VMEM((BPW * S,), jnp.int32),
        pltpu.VMEM((BPW, D), jnp.float32),
        pltpu.VMEM((2, GI, D), jnp.float32),
        pltpu.VMEM((ACC_E, D), jnp.float32),
        pltpu.SemaphoreType.DMA,
        pltpu.SemaphoreType.DMA,
        pltpu.SemaphoreType.DMA,
    ],
)
def _gather_sum(nodes_hbm, nidx_hbm, feat_hbm, self_hbm, sum_hbm,
                nodes_v, nidx_v, sbuf, rbuf, acc, ssem, rsem0, rsem1):
    wid = lax.axis_index("s") * NC + lax.axis_index("c")
    base = wid * BPW

    pltpu.sync_copy(nodes_hbm.at[pl.ds(base, BPW)], nodes_v)
    pltpu.sync_copy(nidx_hbm.at[pl.ds(base * S, BPW * S)], nidx_v)

    for c in range(BPW // SELF_CH):
        pltpu.async_copy(
            feat_hbm.at[nodes_v.at[pl.ds(c * SELF_CH, SELF_CH)]],
            sbuf.at[pl.ds(c * SELF_CH, SELF_CH)], ssem)

    rsems = (rsem0, rsem1)

    def fire(g, slot):
        pltpu.async_copy(
            feat_hbm.at[nidx_v.at[pl.ds(g * GI, GI)]], rbuf.at[slot],
            rsems[slot])

    def wait(slot):
        pltpu.make_async_copy(
            feat_hbm.at[nidx_v.at[pl.ds(0, GI)]], rbuf.at[slot],
            rsems[slot]).wait()

    fire(0, 0)
    fire(1, 1)

    def body(i, carry):
        for b in range(2):
            g = 2 * i + b
            wait(b)
            row0 = (g % (ACC_E // G)) * G
            for e in range(G):
                for v in range(D // L):
                    sl = pl.ds(v * L, L)
                    r = rbuf[b, e * S, sl]
                    for n in range(1, S):
                        r = r + rbuf[b, e * S + n, sl]
                    acc[row0 + e, sl] = r

            @pl.when(g + 2 < NGRP)
            def _():
                fire(g + 2, b)

            if b == 1:
                @pl.when(i % 4 == 3)
                def _():
                    blk = i // 4
                    pltpu.sync_copy(
                        acc, sum_hbm.at[pl.ds(base + blk * ACC_E, ACC_E)])
        return carry

    lax.fori_loop(0, NGRP // 2, body, 0)

    for c in range(BPW // SELF_CH):
        pltpu.make_async_copy(
            feat_hbm.at[nodes_v.at[pl.ds(c * SELF_CH, SELF_CH)]],
            sbuf.at[pl.ds(c * SELF_CH, SELF_CH)], ssem).wait()
    pltpu.sync_copy(sbuf, self_hbm.at[pl.ds(base, BPW)])


BM = 2048


def _mlp_body(self_ref, sum_ref, w1a_ref, w1b_ref, b1_ref, w2_ref, b2_ref,
              o_ref):
    dims = (((1,), (0,)), ((), ()))
    x = self_ref[...]
    m = sum_ref[...] * jnp.float32(1.0 / S)
    h = (lax.dot_general(x, w1a_ref[...], dims,
                         preferred_element_type=jnp.float32,
                         precision=lax.Precision.HIGHEST)
         + lax.dot_general(m, w1b_ref[...], dims,
                           preferred_element_type=jnp.float32,
                           precision=lax.Precision.HIGHEST)
         + b1_ref[...])
    h = jnp.where(h >= 0, h, jnp.float32(0.03) * h)
    o = lax.dot_general(h, w2_ref[...], dims,
                        preferred_element_type=jnp.float32,
                        precision=lax.Precision.HIGHEST) + b2_ref[...]
    o_ref[...] = jnp.tanh(o)


def _mlp(self_f, sum_f, w1a, w1b, b1, w2, b2):
    return pl.pallas_call(
        _mlp_body,
        grid=(B // BM,),
        in_specs=[
            pl.BlockSpec((BM, D), lambda i: (i, 0)),
            pl.BlockSpec((BM, D), lambda i: (i, 0)),
            pl.BlockSpec((D, D), lambda i: (0, 0)),
            pl.BlockSpec((D, D), lambda i: (0, 0)),
            pl.BlockSpec((1, D), lambda i: (0, 0)),
            pl.BlockSpec((D, D), lambda i: (0, 0)),
            pl.BlockSpec((1, D), lambda i: (0, 0)),
        ],
        out_specs=pl.BlockSpec((BM, D), lambda i: (i, 0)),
        out_shape=jax.ShapeDtypeStruct((B, D), jnp.float32),
    )(self_f, sum_f, w1a, w1b, b1, w2, b2)


def kernel(nodes, neigh_idx, features, W1, b1, W2, b2):
    nflat = neigh_idx.reshape(-1)
    self_f, sum_f = _gather_sum(nodes, nflat, features)
    w1a = W1[:D]
    w1b = W1[D:]
    return _mlp(self_f, sum_f, w1a, w1b, b1.reshape(1, D), W2,
                b2.reshape(1, D))


# speedup vs baseline: 3.2424x; 2.0764x over previous
import functools

import jax
import jax.numpy as jnp
from jax import lax
from jax.experimental import pallas as pl
from jax.experimental.pallas import tpu as pltpu
from jax.experimental.pallas import tpu_sc as plsc

B = 16384
D = 128
S = 10
L = 16
NC = 2
NS = 16
NW = NC * NS
BPW = B // NW
G = 8
GI = G * S
NGRP = BPW // G
ACC_E = 64
SELF_CH = 128

_mesh = plsc.VectorSubcoreMesh(core_axis_name="c", subcore_axis_name="s")


@functools.partial(
    pl.kernel,
    out_type=(
        jax.ShapeDtypeStruct((B, D), jnp.float32),
        jax.ShapeDtypeStruct((B, D), jnp.float32),
    ),
    mesh=_mesh,
    scratch_types=[
        pltpu.VMEM((BPW,), jnp.int32),
        pltpu.VMEM((BPW * S,), jnp.int32),
        pltpu.VMEM((BPW, D), jnp.float32),
        pltpu.VMEM((2, GI, D), jnp.float32),
        pltpu.VMEM((ACC_E, D), jnp.float32),
        pltpu.SemaphoreType.DMA,
        pltpu.SemaphoreType.DMA,
        pltpu.SemaphoreType.DMA,
    ],
)
def _gather_sum(nodes_hbm, nidx_hbm, feat_hbm, self_hbm, sum_hbm,
                nodes_v, nidx_v, sbuf, rbuf, acc, ssem, rsem0, rsem1):
    wid = lax.axis_index("s") * NC + lax.axis_index("c")
    base = wid * BPW

    pltpu.sync_copy(nodes_hbm.at[pl.ds(base, BPW)], nodes_v)
    pltpu.sync_copy(nidx_hbm.at[pl.ds(base * S, BPW * S)], nidx_v)

    for c in range(BPW // SELF_CH):
        pltpu.async_copy(
            feat_hbm.at[nodes_v.at[pl.ds(c * SELF_CH, SELF_CH)]],
            sbuf.at[pl.ds(c * SELF_CH, SELF_CH)], ssem)

    rsems = (rsem0, rsem1)

    def fire(g, slot):
        pltpu.async_copy(
            feat_hbm.at[nidx_v.at[pl.ds(g * GI, GI)]], rbuf.at[slot],
            rsems[slot])

    def wait(slot):
        pltpu.make_async_copy(
            feat_hbm.at[nidx_v.at[pl.ds(0, GI)]], rbuf.at[slot],
            rsems[slot]).wait()

    fire(0, 0)
    fire(1, 1)

    def body(i, carry):
        for b in range(2):
            g = 2 * i + b
            wait(b)
            row0 = (g % (ACC_E // G)) * G
            pend = None
            for e in range(G):
                for v in range(D // L):
                    sl = pl.ds(v * L, L)
                    vals = [rbuf[b, e * S + n, sl] for n in range(S)]
                    if pend is not None:
                        p_e, p_sl, p_vals = pend
                        while len(p_vals) > 1:
                            nxt = [p_vals[2 * k] + p_vals[2 * k + 1]
                                   for k in range(len(p_vals) // 2)]
                            if len(p_vals) % 2:
                                nxt.append(p_vals[-1])
                            p_vals = nxt
                        acc[row0 + p_e, p_sl] = p_vals[0]
                    pend = (e, sl, vals)
            p_e, p_sl, p_vals = pend
            while len(p_vals) > 1:
                nxt = [p_vals[2 * k] + p_vals[2 * k + 1]
                       for k in range(len(p_vals) // 2)]
                if len(p_vals) % 2:
                    nxt.append(p_vals[-1])
                p_vals = nxt
            acc[row0 + p_e, p_sl] = p_vals[0]

            @pl.when(g + 2 < NGRP)
            def _():
                fire(g + 2, b)

            if b == 1:
                @pl.when(i % 4 == 3)
                def _():
                    blk = i // 4
                    pltpu.sync_copy(
                        acc, sum_hbm.at[pl.ds(base + blk * ACC_E, ACC_E)])
        return carry

    lax.fori_loop(0, NGRP // 2, body, 0)

    for c in range(BPW // SELF_CH):
        pltpu.make_async_copy(
            feat_hbm.at[nodes_v.at[pl.ds(c * SELF_CH, SELF_CH)]],
            sbuf.at[pl.ds(c * SELF_CH, SELF_CH)], ssem).wait()
    pltpu.sync_copy(sbuf, self_hbm.at[pl.ds(base, BPW)])


BM = 2048


def _mlp_body(self_ref, sum_ref, w1a_ref, w1b_ref, b1_ref, w2_ref, b2_ref,
              o_ref):
    dims = (((1,), (0,)), ((), ()))
    x = self_ref[...]
    m = sum_ref[...] * jnp.float32(1.0 / S)
    h = (lax.dot_general(x, w1a_ref[...], dims,
                         preferred_element_type=jnp.float32,
                         precision=None)
         + lax.dot_general(m, w1b_ref[...], dims,
                           preferred_element_type=jnp.float32,
                           precision=None)
         + b1_ref[...])
    h = jnp.where(h >= 0, h, jnp.float32(0.03) * h)
    o = lax.dot_general(h, w2_ref[...], dims,
                        preferred_element_type=jnp.float32,
                        precision=None) + b2_ref[...]
    o_ref[...] = jnp.tanh(o)


def _mlp(self_f, sum_f, w1a, w1b, b1, w2, b2):
    return pl.pallas_call(
        _mlp_body,
        grid=(B // BM,),
        in_specs=[
            pl.BlockSpec((BM, D), lambda i: (i, 0)),
            pl.BlockSpec((BM, D), lambda i: (i, 0)),
            pl.BlockSpec((D, D), lambda i: (0, 0)),
            pl.BlockSpec((D, D), lambda i: (0, 0)),
            pl.BlockSpec((1, D), lambda i: (0, 0)),
            pl.BlockSpec((D, D), lambda i: (0, 0)),
            pl.BlockSpec((1, D), lambda i: (0, 0)),
        ],
        out_specs=pl.BlockSpec((BM, D), lambda i: (i, 0)),
        out_shape=jax.ShapeDtypeStruct((B, D), jnp.float32),
    )(self_f, sum_f, w1a, w1b, b1, w2, b2)


def kernel(nodes, neigh_idx, features, W1, b1, W2, b2):
    nflat = neigh_idx.reshape(-1)
    self_f, sum_f = _gather_sum(nodes, nflat, features)
    w1a = W1[:D]
    w1b = W1[D:]
    return _mlp(self_f, sum_f, w1a, w1b, b1.reshape(1, D), W2,
                b2.reshape(1, D))
